# Initial kernel scaffold; baseline (speedup 1.0000x reference)
import jax
import jax.numpy as jnp
from jax.experimental import pallas as pl


def kernel(inputs, W):
    raise NotImplementedError("write your pallas kernel here")


# speedup vs baseline: 2.5102x; 2.5102x over previous
import jax
import jax.numpy as jnp
from jax.experimental import pallas as pl

_COMMITMENT = 0.25
_EPS = 1e-12


def _vq_tile_kernel(x_ref, w_ref, q_ref, loss_ref, idx_ref):
    w = w_ref[...]
    wn2 = jnp.sum(w * w, axis=1, keepdims=True)
    wn = w / jnp.maximum(jnp.sqrt(wn2), _EPS)
    wsq = jnp.sum(wn * wn, axis=1, keepdims=True)

    x = x_ref[0]
    xn2 = jnp.sum(x * x, axis=0, keepdims=True)
    xn = x / jnp.maximum(jnp.sqrt(xn2), _EPS)
    xsq = jnp.sum(xn * xn, axis=0, keepdims=True)

    dots = jnp.dot(wn, xn, preferred_element_type=jnp.float32)
    dist = (xsq + wsq) - 2.0 * dots

    idx0 = jnp.argmin(dist, axis=0)

    onehot = (jax.lax.broadcasted_iota(jnp.int32, dist.shape, 0)
              == idx0[None, :]).astype(jnp.float32)
    q = jax.lax.dot_general(wn, onehot,
                            dimension_numbers=(((0,), (0,)), ((), ())),
                            preferred_element_type=jnp.float32)

    d = q - xn
    sq = d * d
    loss = jnp.mean(sq + _COMMITMENT * sq, axis=0)

    q_ref[0] = q
    loss_ref[0, 0, :] = loss
    idx_ref[0, 0, :] = (idx0 + 1).astype(jnp.int32)


def kernel(inputs, W):
    B, C, nz, nt, nr = inputs.shape
    S = nz * nt * nr
    x3 = inputs.reshape(B, C, S)
    W1 = W[1:]
    K = W1.shape[0]

    T = 512
    grid = (B, S // T)

    q3, loss3, idx3 = pl.pallas_call(
        _vq_tile_kernel,
        grid=grid,
        in_specs=[
            pl.BlockSpec((1, C, T), lambda b, t: (b, 0, t)),
            pl.BlockSpec((K, C), lambda b, t: (0, 0)),
        ],
        out_specs=[
            pl.BlockSpec((1, C, T), lambda b, t: (b, 0, t)),
            pl.BlockSpec((1, 1, T), lambda b, t: (b, 0, t)),
            pl.BlockSpec((1, 1, T), lambda b, t: (b, 0, t)),
        ],
        out_shape=[
            jax.ShapeDtypeStruct((B, C, S), jnp.float32),
            jax.ShapeDtypeStruct((B, 1, S), jnp.float32),
            jax.ShapeDtypeStruct((B, 1, S), jnp.int32),
        ],
    )(x3, W1)

    quantized_out = q3.reshape(B, C, nz, nt, nr)
    vq_loss_spatial = loss3.reshape(B, nz, nt, nr)
    spatial_indices = idx3.reshape(B, nz, nt, nr)
    return quantized_out, vq_loss_spatial, spatial_indices


# speedup vs baseline: 3.1536x; 1.2563x over previous
import jax
import jax.numpy as jnp
from jax.experimental import pallas as pl

_COMMITMENT = 0.25
_EPS = 1e-12


def _wnorm_kernel(w_ref, wn_ref):
    w = w_ref[...]
    wn2 = jnp.sum(w * w, axis=1, keepdims=True)
    wn_ref[...] = w / jnp.maximum(jnp.sqrt(wn2), _EPS)


def _vq_tile_kernel(x_ref, w_ref, q_ref, loss_ref, idx_ref):
    wn = w_ref[...]
    wsq = jnp.sum(wn * wn, axis=1, keepdims=True)

    x = x_ref[0]
    xn2 = jnp.sum(x * x, axis=0, keepdims=True)
    xn = x / jnp.maximum(jnp.sqrt(xn2), _EPS)
    xsq = jnp.sum(xn * xn, axis=0, keepdims=True)

    dots = jnp.dot(wn, xn, preferred_element_type=jnp.float32)
    dist = (xsq + wsq) - 2.0 * dots

    idx0 = jnp.argmin(dist, axis=0)

    onehot = (jax.lax.broadcasted_iota(jnp.int32, dist.shape, 0)
              == idx0[None, :]).astype(jnp.float32)
    q = jax.lax.dot_general(wn, onehot,
                            dimension_numbers=(((0,), (0,)), ((), ())),
                            preferred_element_type=jnp.float32)

    d = q - xn
    sq = d * d
    loss = jnp.mean(sq + _COMMITMENT * sq, axis=0)

    q_ref[0] = q
    loss_ref[0, 0, :] = loss
    idx_ref[0, 0, :] = (idx0 + 1).astype(jnp.int32)


def kernel(inputs, W):
    B, C, nz, nt, nr = inputs.shape
    S = nz * nt * nr
    x3 = inputs.reshape(B, C, S)
    W1 = W[1:]
    K = W1.shape[0]

    wn = pl.pallas_call(
        _wnorm_kernel,
        out_shape=jax.ShapeDtypeStruct((K, C), jnp.float32),
    )(W1)

    T = 1024
    grid = (B, S // T)

    q3, loss3, idx3 = pl.pallas_call(
        _vq_tile_kernel,
        grid=grid,
        in_specs=[
            pl.BlockSpec((1, C, T), lambda b, t: (b, 0, t)),
            pl.BlockSpec((K, C), lambda b, t: (0, 0)),
        ],
        out_specs=[
            pl.BlockSpec((1, C, T), lambda b, t: (b, 0, t)),
            pl.BlockSpec((1, 1, T), lambda b, t: (b, 0, t)),
            pl.BlockSpec((1, 1, T), lambda b, t: (b, 0, t)),
        ],
        out_shape=[
            jax.ShapeDtypeStruct((B, C, S), jnp.float32),
            jax.ShapeDtypeStruct((B, 1, S), jnp.float32),
            jax.ShapeDtypeStruct((B, 1, S), jnp.int32),
        ],
    )(x3, wn)

    quantized_out = q3.reshape(B, C, nz, nt, nr)
    vq_loss_spatial = loss3.reshape(B, nz, nt, nr)
    spatial_indices = idx3.reshape(B, nz, nt, nr)
    return quantized_out, vq_loss_spatial, spatial_indices
